# baseline (speedup 1.0000x reference)
import functools

import jax
import jax.numpy as jnp
from jax import lax
from jax.experimental import pallas as pl
from jax.experimental.pallas import tpu as pltpu
from jax.experimental.pallas import tpu_sc as plsc

DIM = 64
C = 256


def _emb_call(tk_t, wt128, seq, batch):
    info = plsc.get_sparse_core_info()
    nc, ns = info.num_cores, info.num_subcores
    nw = nc * ns
    units_per_s = batch // C
    n_units = seq * units_per_s
    upw = n_units // nw

    mesh = plsc.VectorSubcoreMesh(core_axis_name="c", subcore_axis_name="s")

    @functools.partial(
        pl.kernel,
        mesh=mesh,
        out_type=jax.ShapeDtypeStruct((seq, DIM, batch), jnp.float32),
        scratch_types=[
            pltpu.VMEM((C,), jnp.int32),
            pltpu.VMEM((C,), jnp.int32),
            pltpu.VMEM((C,), jnp.int32),
            pltpu.VMEM((C, 128), jnp.float32),
            pltpu.VMEM((DIM, C), jnp.float32),
            pltpu.SemaphoreType.DMA,
        ],
        compiler_params=pltpu.CompilerParams(needs_layout_passes=False),
    )
    def emb(tk_hbm, wt_hbm, out_hbm, idxv, gv, hv, buf, outc, sem):
        wid = lax.axis_index("s") * nc + lax.axis_index("c")
        u0 = wid * upw
        lanes = lax.iota(jnp.int32, 16)

        def unit_body(k, carry):
            u = u0 + k
            s = u // units_per_s
            b0 = (u % units_per_s) * C
            pltpu.sync_copy(tk_hbm.at[s, pl.ds(b0, C)], idxv)
            pltpu.async_copy(wt_hbm.at[idxv], buf, sem).wait()
            rvecs = [jb * 16 + lanes for jb in range(C // 16)]

            def dbody(d, dcarry):
                cols = jnp.full((16,), 0, jnp.int32) + d
                for jb in range(C // 16):
                    val = plsc.load_gather(buf, [rvecs[jb], cols])
                    outc[d, pl.ds(jb * 16, 16)] = val
                return dcarry

            lax.fori_loop(0, DIM, dbody, 0)
            pltpu.sync_copy(outc, out_hbm.at[s, :, pl.ds(b0, C)])
            return carry

        lax.fori_loop(0, upw, unit_body, 0)

    return emb(tk_t, wt128)


def kernel(tokens, weight):
    b, s = tokens.shape
    tk_t = tokens.astype(jnp.int32).T
    wt128 = jnp.pad(weight, ((0, 0), (0, 128 - weight.shape[1])))
    out_t = _emb_call(tk_t, wt128, s, b)
    return jnp.transpose(out_t, (2, 0, 1))


# speedup vs baseline: 2.1004x; 2.1004x over previous
import functools

import jax
import jax.numpy as jnp
from jax import lax
from jax.experimental import pallas as pl
from jax.experimental.layout import Format, Layout, with_layout_constraint
from jax.experimental.pallas import tpu as pltpu
from jax.experimental.pallas import tpu_sc as plsc

DIM = 64
NBUF = 4
CHUNK = 320


def _emb_call(idx, weight, num_rows):
    info = plsc.get_sparse_core_info()
    nc, ns = info.num_cores, info.num_subcores
    nw = nc * ns
    rows_per_w = num_rows // nw
    n_chunks = rows_per_w // CHUNK
    n_outer = n_chunks // NBUF

    mesh = plsc.VectorSubcoreMesh(core_axis_name="c", subcore_axis_name="s")

    @functools.partial(
        pl.kernel,
        mesh=mesh,
        out_type=jax.ShapeDtypeStruct((num_rows, DIM), jnp.float32),
        scratch_types=[
            pltpu.VMEM((rows_per_w,), jnp.int32),
            pltpu.VMEM((NBUF, CHUNK, DIM), jnp.float32),
            [pltpu.SemaphoreType.DMA] * NBUF,
            [pltpu.SemaphoreType.DMA] * NBUF,
        ],
        compiler_params=pltpu.CompilerParams(use_tc_tiling_on_sc=False),
    )
    def emb(idx_hbm, table_hbm, out_hbm, idx_v, rows_v, gsems, ssems):
        wid = lax.axis_index("s") * nc + lax.axis_index("c")
        base = wid * rows_per_w
        pltpu.sync_copy(idx_hbm.at[pl.ds(base, rows_per_w)], idx_v)

        def outer(g, carry):
            for b in range(NBUF):
                i = g * NBUF + b

                @pl.when(g > 0)
                def _wait_store():
                    off_prev = base + (i - NBUF) * CHUNK
                    pltpu.make_async_copy(
                        rows_v.at[b], out_hbm.at[pl.ds(off_prev, CHUNK)], ssems[b]
                    ).wait()

                pltpu.async_copy(
                    table_hbm.at[idx_v.at[pl.ds(i * CHUNK, CHUNK)]],
                    rows_v.at[b],
                    gsems[b],
                )
            for b in range(NBUF):
                i = g * NBUF + b
                off = base + i * CHUNK
                pltpu.make_async_copy(
                    table_hbm.at[idx_v.at[pl.ds(i * CHUNK, CHUNK)]],
                    rows_v.at[b],
                    gsems[b],
                ).wait()
                pltpu.async_copy(
                    rows_v.at[b], out_hbm.at[pl.ds(off, CHUNK)], ssems[b]
                )
            return carry

        lax.fori_loop(0, n_outer, outer, 0)

        for b in range(NBUF):
            i = (n_outer - 1) * NBUF + b
            off = base + i * CHUNK
            pltpu.make_async_copy(
                rows_v.at[b], out_hbm.at[pl.ds(off, CHUNK)], ssems[b]
            ).wait()

    return emb(idx, weight)


def kernel(tokens, weight):
    b, s = tokens.shape
    num_rows = b * s
    idx = tokens.reshape(num_rows).astype(jnp.int32)
    wl = with_layout_constraint(
        weight, Layout(major_to_minor=(0, 1), tiling=((16,),))
    )
    out = _emb_call(idx, wl, num_rows)
    out3 = out.reshape(b, s, DIM)
    return with_layout_constraint(
        out3, Layout(major_to_minor=(1, 2, 0), tiling=((8, 128),))
    )
